# Initial kernel scaffold; baseline (speedup 1.0000x reference)
import jax
import jax.numpy as jnp
from jax.experimental import pallas as pl


def kernel(x, table, W, b):
    raise NotImplementedError("write your pallas kernel here")


# speedup vs baseline: 17.8817x; 17.8817x over previous
import functools

import jax
import jax.numpy as jnp
from jax import lax
from jax.experimental import pallas as pl
from jax.experimental.pallas import tpu as pltpu
from jax.experimental.pallas import tpu_sc as plsc

B = 4096
L = 200
D = 32
N_TOTAL = B * L

NC = 2
NS = 16
NW = NC * NS
PER_W = N_TOTAL // NW
CHUNK = 128
N_CHUNK = PER_W // CHUNK


def _sc_gather(table, idx3):
    mesh = plsc.VectorSubcoreMesh(core_axis_name="c", subcore_axis_name="s")

    @functools.partial(
        pl.kernel,
        out_type=jax.ShapeDtypeStruct((N_TOTAL, D), jnp.float32),
        mesh=mesh,
        scratch_types=[
            pltpu.VMEM((N_CHUNK, CHUNK), jnp.int32),
            pltpu.VMEM((CHUNK, D), jnp.float32),
            pltpu.VMEM((CHUNK, D), jnp.float32),
            pltpu.SemaphoreType.DMA,
            pltpu.SemaphoreType.DMA,
        ],
        compiler_params=pltpu.CompilerParams(use_tc_tiling_on_sc=False),
    )
    def k(table_hbm, idx_hbm, out_hbm, idx_v, rows_a, rows_b, sem_a, sem_b):
        wid = lax.axis_index("s") * NC + lax.axis_index("c")
        base = wid * PER_W
        pltpu.sync_copy(idx_hbm.at[wid], idx_v)

        first = pltpu.async_copy(table_hbm.at[idx_v.at[0]], rows_a, sem_a)

        @pl.loop(0, N_CHUNK // 2)
        def _(p):
            j = p * 2
            nxt = pltpu.async_copy(table_hbm.at[idx_v.at[j + 1]], rows_b, sem_b)
            pltpu.make_async_copy(table_hbm.at[idx_v.at[j]], rows_a, sem_a).wait()
            pltpu.sync_copy(rows_a, out_hbm.at[pl.ds(base + j * CHUNK, CHUNK)])

            @pl.when(j + 2 < N_CHUNK)
            def _():
                pltpu.async_copy(table_hbm.at[idx_v.at[j + 2]], rows_a, sem_a)

            pltpu.make_async_copy(table_hbm.at[idx_v.at[j + 1]], rows_b, sem_b).wait()
            pltpu.sync_copy(rows_b, out_hbm.at[pl.ds(base + (j + 1) * CHUNK, CHUNK)])

    return k(table, idx3)


ROWS_BLK = 2048


def _tc_linear_tanh(packed, bd, bias_tile):
    n_rows = packed.shape[0]

    def body(x_ref, bd_ref, b_ref, o_ref):
        acc = jnp.dot(x_ref[...], bd_ref[...],
                      preferred_element_type=jnp.float32)
        o_ref[...] = jnp.tanh(acc + b_ref[...])

    return pl.pallas_call(
        body,
        grid=(n_rows // ROWS_BLK,),
        in_specs=[
            pl.BlockSpec((ROWS_BLK, 128), lambda i: (i, 0)),
            pl.BlockSpec((128, 128), lambda i: (0, 0)),
            pl.BlockSpec((1, 128), lambda i: (0, 0)),
        ],
        out_specs=pl.BlockSpec((ROWS_BLK, 128), lambda i: (i, 0)),
        out_shape=jax.ShapeDtypeStruct((n_rows, 128), jnp.float32),
    )(packed, bd, bias_tile)


def kernel(x, table, W, b):
    idx3 = x.astype(jnp.int32).reshape(NW, N_CHUNK, CHUNK)
    gathered = _sc_gather(table, idx3)

    wt = W.T
    bd = jnp.zeros((128, 128), jnp.float32)
    for q in range(4):
        bd = bd.at[q * D:(q + 1) * D, q * D:(q + 1) * D].set(wt)
    bias_tile = jnp.tile(b, 4).reshape(1, 128)

    packed = gathered.reshape(N_TOTAL // 4, 128)
    out = _tc_linear_tanh(packed, bd, bias_tile)
    return out.reshape(B, L, D)
